# baseline (speedup 1.0000x reference)
import functools

import jax
import jax.numpy as jnp
from jax.experimental import pallas as pl
from jax.experimental.pallas import tpu as pltpu

FEATURE_DIM = 4096
TEXT_DIM = 1024
PROJ = 128
NUM_EXPERTS = 8
TOKENS = 8192
TILE = 1024


def _router_kernel(scale_ref, f_ref, t_ref, fmu_ref, tmu_ref, cmu_ref,
                   probs_ref, logits_ref):
    fp = jnp.dot(f_ref[...], fmu_ref[...], preferred_element_type=jnp.float32)
    tp = jnp.dot(t_ref[...], tmu_ref[...], preferred_element_type=jnp.float32)
    logits = (
        jnp.dot(fp, cmu_ref[:PROJ, :], preferred_element_type=jnp.float32)
        + jnp.dot(tp, cmu_ref[PROJ:, :], preferred_element_type=jnp.float32)
    ) * scale_ref[0]
    logits_ref[...] = logits
    m = jnp.max(logits, axis=1, keepdims=True)
    e = jnp.exp(logits - m)
    probs_ref[...] = e / jnp.sum(e, axis=1, keepdims=True)


@functools.partial(jax.jit, static_argnames=())
def kernel(feature, text_embedding, feature_mu, text_mu, combined_mu,
           temperature):
    scale = 1.0 / jnp.clip(temperature, 0.1, None)
    grid = (TOKENS // TILE,)
    probs, logits = pl.pallas_call(
        _router_kernel,
        grid_spec=pltpu.PrefetchScalarGridSpec(
            num_scalar_prefetch=0,
            grid=grid,
            in_specs=[
                pl.BlockSpec(memory_space=pltpu.SMEM),
                pl.BlockSpec((TILE, FEATURE_DIM), lambda i: (i, 0)),
                pl.BlockSpec((TILE, TEXT_DIM), lambda i: (i, 0)),
                pl.BlockSpec((FEATURE_DIM, PROJ), lambda i: (0, 0)),
                pl.BlockSpec((TEXT_DIM, PROJ), lambda i: (0, 0)),
                pl.BlockSpec((2 * PROJ, NUM_EXPERTS), lambda i: (0, 0)),
            ],
            out_specs=[
                pl.BlockSpec((TILE, NUM_EXPERTS), lambda i: (i, 0)),
                pl.BlockSpec((TILE, NUM_EXPERTS), lambda i: (i, 0)),
            ],
        ),
        out_shape=[
            jax.ShapeDtypeStruct((TOKENS, NUM_EXPERTS), jnp.float32),
            jax.ShapeDtypeStruct((TOKENS, NUM_EXPERTS), jnp.float32),
        ],
        compiler_params=pltpu.CompilerParams(
            dimension_semantics=("arbitrary",),
        ),
    )(scale, feature, text_embedding, feature_mu, text_mu, combined_mu)
    return probs, logits


# speedup vs baseline: 1.0309x; 1.0309x over previous
import functools

import jax
import jax.numpy as jnp
from jax.experimental import pallas as pl
from jax.experimental.pallas import tpu as pltpu

FEATURE_DIM = 4096
TEXT_DIM = 1024
PROJ = 128
NUM_EXPERTS = 8
TOKENS = 8192
TILE = 512


def _router_kernel(scale_ref, f_ref, t_ref, fmu_ref, tmu_ref, cmu_ref,
                   probs_ref, logits_ref):
    fp = jnp.dot(f_ref[...], fmu_ref[...], preferred_element_type=jnp.float32)
    tp = jnp.dot(t_ref[...], tmu_ref[...], preferred_element_type=jnp.float32)
    logits = (
        jnp.dot(fp, cmu_ref[:PROJ, :], preferred_element_type=jnp.float32)
        + jnp.dot(tp, cmu_ref[PROJ:, :], preferred_element_type=jnp.float32)
    ) * scale_ref[0]
    logits_ref[...] = logits
    m = jnp.max(logits, axis=1, keepdims=True)
    e = jnp.exp(logits - m)
    probs_ref[...] = e / jnp.sum(e, axis=1, keepdims=True)


@functools.partial(jax.jit, static_argnames=())
def kernel(feature, text_embedding, feature_mu, text_mu, combined_mu,
           temperature):
    scale = 1.0 / jnp.clip(temperature, 0.1, None)
    grid = (TOKENS // TILE,)
    probs, logits = pl.pallas_call(
        _router_kernel,
        grid_spec=pltpu.PrefetchScalarGridSpec(
            num_scalar_prefetch=0,
            grid=grid,
            in_specs=[
                pl.BlockSpec(memory_space=pltpu.SMEM),
                pl.BlockSpec((TILE, FEATURE_DIM), lambda i: (i, 0)),
                pl.BlockSpec((TILE, TEXT_DIM), lambda i: (i, 0)),
                pl.BlockSpec((FEATURE_DIM, PROJ), lambda i: (0, 0)),
                pl.BlockSpec((TEXT_DIM, PROJ), lambda i: (0, 0)),
                pl.BlockSpec((2 * PROJ, NUM_EXPERTS), lambda i: (0, 0)),
            ],
            out_specs=[
                pl.BlockSpec((TILE, NUM_EXPERTS), lambda i: (i, 0)),
                pl.BlockSpec((TILE, NUM_EXPERTS), lambda i: (i, 0)),
            ],
        ),
        out_shape=[
            jax.ShapeDtypeStruct((TOKENS, NUM_EXPERTS), jnp.float32),
            jax.ShapeDtypeStruct((TOKENS, NUM_EXPERTS), jnp.float32),
        ],
        compiler_params=pltpu.CompilerParams(
            dimension_semantics=("arbitrary",),
        ),
    )(scale, feature, text_embedding, feature_mu, text_mu, combined_mu)
    return probs, logits
